# Initial kernel scaffold; baseline (speedup 1.0000x reference)
import functools

import jax
import jax.numpy as jnp
import numpy as np
from jax import lax
from jax.experimental import pallas as pl
from jax.experimental.pallas import tpu as pltpu
from jax.experimental.pallas import tpu_sc as plsc

TABLE_SZ = 524288
FEATURE_DIM = 2
NUM_LEVELS = 16
MIN_RES = 16
B_GROWTH = 1.38
BATCH = 262144

NC, NS = 2, 16
NW = NC * NS
PTS_PER_W = BATCH // NW
CHUNK = 1024
NCHUNK = PTS_PER_W // CHUNK
GROUPS = CHUNK // 16

_MASK = TABLE_SZ - 1
_C1 = np.int32(np.uint32(2654435761).astype(np.int32))
_C2 = np.int32(805459861)
_RES = [float(np.floor(np.float32(MIN_RES * (B_GROWTH ** l)))) for l in range(NUM_LEVELS)]

_CORNERS = [
    (0, 0, 0), (1, 0, 0), (1, 1, 0), (0, 1, 0),
    (0, 0, 1), (1, 0, 1), (1, 1, 1), (0, 1, 1),
]


def _body(x0, x1, x2, table, out, xs_v, ys_v, zs_v, idx_v, rows_v, out_v, sem):
    wid = lax.axis_index("s") * NC + lax.axis_index("c")
    lane = lax.iota(jnp.int32, 16)
    zeros16 = jnp.zeros((16,), jnp.int32)
    ones16 = jnp.ones((16,), jnp.int32)

    def chunk_body(ci, carry):
        base = wid * PTS_PER_W + ci * CHUNK
        pltpu.sync_copy(x0.at[pl.ds(base, CHUNK)], xs_v)
        pltpu.sync_copy(x1.at[pl.ds(base, CHUNK)], ys_v)
        pltpu.sync_copy(x2.at[pl.ds(base, CHUNK)], zs_v)

        for lvl in range(NUM_LEVELS):
            res = np.float32(_RES[lvl])
            off = np.int32(lvl * TABLE_SZ)

            def build(g, c2, _res=res, _off=off):
                xv = xs_v[pl.ds(g * 16, 16)]
                yv = ys_v[pl.ds(g * 16, 16)]
                zv = zs_v[pl.ds(g * 16, 16)]
                spx = xv * _res
                spy = yv * _res
                spz = zv * _res
                lx = spx.astype(jnp.int32)
                ly = spy.astype(jnp.int32)
                lz = spz.astype(jnp.int32)
                hx = lx + (spx > lx.astype(jnp.float32)).astype(jnp.int32)
                hy = ly + (spy > ly.astype(jnp.float32)).astype(jnp.int32)
                hz = lz + (spz > lz.astype(jnp.float32)).astype(jnp.int32)
                ax = (lx, hx)
                by = (ly * _C1, hy * _C1)
                cz = (lz * _C2, hz * _C2)
                gbase = g * 128
                for c, (i, j, k) in enumerate(_CORNERS):
                    h = ((ax[i] ^ by[j] ^ cz[k]) & _MASK) + _off
                    idx_v[pl.ds(gbase + c * 16, 16)] = h
                return c2

            lax.fori_loop(0, GROUPS, build, 0)
            pltpu.async_copy(table.at[idx_v], rows_v, sem).wait()

            col0 = jnp.full((16,), 2 * lvl, jnp.int32)
            col1 = jnp.full((16,), 2 * lvl + 1, jnp.int32)

            def interp(g, c2, _col0=col0, _col1=col1):
                xv = xs_v[pl.ds(g * 16, 16)]
                yv = ys_v[pl.ds(g * 16, 16)]
                zv = zs_v[pl.ds(g * 16, 16)]
                xw = xv - xv.astype(jnp.int32).astype(jnp.float32)
                yw = yv - yv.astype(jnp.int32).astype(jnp.float32)
                zw = zv - zv.astype(jnp.int32).astype(jnp.float32)
                gbase = g * 128
                f0 = []
                f1 = []
                for c in range(8):
                    rid = lane + (gbase + c * 16)
                    f0.append(plsc.load_gather(rows_v, [rid, zeros16]))
                    f1.append(plsc.load_gather(rows_v, [rid, ones16]))
                orow = g * 16 + lane
                for ch, f in ((0, f0), (1, f1)):
                    c00 = f[0] * (1.0 - xw) + f[1] * xw
                    c01 = f[4] * (1.0 - xw) + f[5] * xw
                    c10 = f[3] * (1.0 - xw) + f[2] * xw
                    c11 = f[7] * (1.0 - xw) + f[6] * xw
                    c0 = c00 * (1.0 - yw) + c10 * yw
                    c1 = c01 * (1.0 - yw) + c11 * yw
                    val = c0 * (1.0 - zw) + c1 * zw
                    plsc.store_scatter(out_v, [orow, _col0 if ch == 0 else _col1], val)
                return c2

            lax.fori_loop(0, GROUPS, interp, 0)

        pltpu.sync_copy(out_v, out.at[pl.ds(base, CHUNK), :])
        return carry

    lax.fori_loop(0, NCHUNK, chunk_body, 0)


_mesh = plsc.VectorSubcoreMesh(core_axis_name="c", subcore_axis_name="s")

_hash_enc = functools.partial(
    pl.kernel,
    out_type=jax.ShapeDtypeStruct((BATCH, 2 * NUM_LEVELS), jnp.float32),
    mesh=_mesh,
    scratch_types=[
        pltpu.VMEM((CHUNK,), jnp.float32),
        pltpu.VMEM((CHUNK,), jnp.float32),
        pltpu.VMEM((CHUNK,), jnp.float32),
        pltpu.VMEM((CHUNK * 8,), jnp.int32),
        pltpu.VMEM((CHUNK * 8, FEATURE_DIM), jnp.float32),
        pltpu.VMEM((CHUNK, 2 * NUM_LEVELS), jnp.float32),
        pltpu.SemaphoreType.DMA,
    ],
)(_body)


def kernel(x, tables):
    xt = x.T
    table = tables.reshape(NUM_LEVELS * TABLE_SZ, FEATURE_DIM)
    return _hash_enc(xt[0], xt[1], xt[2], table)


# speedup vs baseline: 22.7182x; 22.7182x over previous
import functools

import jax
import jax.numpy as jnp
import numpy as np
from jax import lax
from jax.experimental import pallas as pl
from jax.experimental.pallas import tpu as pltpu
from jax.experimental.pallas import tpu_sc as plsc

TABLE_SZ = 524288
FEATURE_DIM = 2
NUM_LEVELS = 16
MIN_RES = 16
B_GROWTH = 1.38
BATCH = 262144

NC, NS = 2, 16
NW = NC * NS
PTS_PER_W = BATCH // NW
CHUNK = 1024
NCHUNK = PTS_PER_W // CHUNK
GROUPS = CHUNK // 16
OUT_W = 2 * NUM_LEVELS

_MASK = TABLE_SZ - 1
_C1 = np.int32(np.uint32(2654435761).astype(np.int32))
_C2 = np.int32(805459861)
_RES = [float(np.floor(np.float32(MIN_RES * (B_GROWTH ** l)))) for l in range(NUM_LEVELS)]

_CORNERS = [
    (0, 0, 0), (1, 0, 0), (1, 1, 0), (0, 1, 0),
    (0, 0, 1), (1, 0, 1), (1, 1, 1), (0, 1, 1),
]


def _body(x0, x1, x2, table, out, xs_v, ys_v, zs_v, idx0_v, idx1_v,
          rows0_v, rows1_v, out_v, sem0, sem1):
    wid = lax.axis_index("s") * NC + lax.axis_index("c")
    lane = lax.iota(jnp.int32, 16)
    lane_w = lane * OUT_W

    def chunk_body(ci, carry):
        base = wid * PTS_PER_W + ci * CHUNK
        pltpu.sync_copy(x0.at[pl.ds(base, CHUNK)], xs_v)
        pltpu.sync_copy(x1.at[pl.ds(base, CHUNK)], ys_v)
        pltpu.sync_copy(x2.at[pl.ds(base, CHUNK)], zs_v)

        for lvl in range(NUM_LEVELS):
            res = np.float32(_RES[lvl])
            off = np.int32(lvl * TABLE_SZ)

            def build(g, c2, _res=res, _off=off):
                xv = xs_v[pl.ds(g * 16, 16)]
                yv = ys_v[pl.ds(g * 16, 16)]
                zv = zs_v[pl.ds(g * 16, 16)]
                spx = xv * _res
                spy = yv * _res
                spz = zv * _res
                lx = spx.astype(jnp.int32)
                ly = spy.astype(jnp.int32)
                lz = spz.astype(jnp.int32)
                hx = jnp.where(spx > lx.astype(jnp.float32), lx + 1, lx)
                hy = jnp.where(spy > ly.astype(jnp.float32), ly + 1, ly)
                hz = jnp.where(spz > lz.astype(jnp.float32), lz + 1, lz)
                ax = (lx, hx)
                by = (ly * _C1, hy * _C1)
                cz = (lz * _C2, hz * _C2)
                gbase = g * 128
                for c, (i, j, k) in enumerate(_CORNERS):
                    h = (((ax[i] ^ by[j] ^ cz[k]) & _MASK) + _off) * 2
                    idx0_v[pl.ds(gbase + c * 16, 16)] = h
                    idx1_v[pl.ds(gbase + c * 16, 16)] = h + 1
                return c2

            lax.fori_loop(0, GROUPS, build, 0)
            cp0 = pltpu.async_copy(table.at[idx0_v], rows0_v, sem0)
            cp1 = pltpu.async_copy(table.at[idx1_v], rows1_v, sem1)
            cp0.wait()
            cp1.wait()

            def interp(g, c2, _lvl=lvl):
                gp = g * 16
                xw = xs_v[pl.ds(gp, 16)]
                yw = ys_v[pl.ds(gp, 16)]
                zw = zs_v[pl.ds(gp, 16)]
                gbase = g * 128
                obase = g * 16 * OUT_W + 2 * _lvl
                for ch, rows in ((0, rows0_v), (1, rows1_v)):
                    f = [rows[pl.ds(gbase + c * 16, 16)] for c in range(8)]
                    c00 = f[0] + xw * (f[1] - f[0])
                    c01 = f[4] + xw * (f[5] - f[4])
                    c10 = f[3] + xw * (f[2] - f[3])
                    c11 = f[7] + xw * (f[6] - f[7])
                    c0 = c00 + yw * (c10 - c00)
                    c1 = c01 + yw * (c11 - c01)
                    val = c0 + zw * (c1 - c0)
                    plsc.store_scatter(out_v, [lane_w + (obase + ch)], val)
                return c2

            lax.fori_loop(0, GROUPS, interp, 0)

        pltpu.sync_copy(out_v, out.at[pl.ds(base * OUT_W, CHUNK * OUT_W)])
        return carry

    lax.fori_loop(0, NCHUNK, chunk_body, 0)


_mesh = plsc.VectorSubcoreMesh(core_axis_name="c", subcore_axis_name="s")

_hash_enc = functools.partial(
    pl.kernel,
    out_type=jax.ShapeDtypeStruct((BATCH * OUT_W,), jnp.float32),
    mesh=_mesh,
    scratch_types=[
        pltpu.VMEM((CHUNK,), jnp.float32),
        pltpu.VMEM((CHUNK,), jnp.float32),
        pltpu.VMEM((CHUNK,), jnp.float32),
        pltpu.VMEM((CHUNK * 8,), jnp.int32),
        pltpu.VMEM((CHUNK * 8,), jnp.int32),
        pltpu.VMEM((CHUNK * 8,), jnp.float32),
        pltpu.VMEM((CHUNK * 8,), jnp.float32),
        pltpu.VMEM((CHUNK * OUT_W,), jnp.float32),
        pltpu.SemaphoreType.DMA,
        pltpu.SemaphoreType.DMA,
    ],
    compiler_params=pltpu.CompilerParams(needs_layout_passes=False),
)(_body)


def kernel(x, tables):
    xt = x.T
    table = tables.reshape(NUM_LEVELS * TABLE_SZ * FEATURE_DIM)
    flat = _hash_enc(xt[0], xt[1], xt[2], table)
    return flat.reshape(BATCH, OUT_W)


# speedup vs baseline: 22.9320x; 1.0094x over previous
import functools

import jax
import jax.numpy as jnp
import numpy as np
from jax import lax
from jax.experimental import pallas as pl
from jax.experimental.pallas import tpu as pltpu
from jax.experimental.pallas import tpu_sc as plsc

TABLE_SZ = 524288
FEATURE_DIM = 2
NUM_LEVELS = 16
MIN_RES = 16
B_GROWTH = 1.38
BATCH = 262144

NC, NS = 2, 16
NW = NC * NS
PTS_PER_W = BATCH // NW
CHUNK = 1024
NCHUNK = PTS_PER_W // CHUNK
GROUPS = CHUNK // 16
OUT_W = 2 * NUM_LEVELS

_MASK = TABLE_SZ - 1
_C1 = np.int32(np.uint32(2654435761).astype(np.int32))
_C2 = np.int32(805459861)
_RES = [float(np.floor(np.float32(MIN_RES * (B_GROWTH ** l)))) for l in range(NUM_LEVELS)]

_CORNERS = [
    (0, 0, 0), (1, 0, 0), (1, 1, 0), (0, 1, 0),
    (0, 0, 1), (1, 0, 1), (1, 1, 1), (0, 1, 1),
]


def _body(x0, x1, x2, table, out, xs_v, ys_v, zs_v,
          idx0_a, idx0_b, idx1_a, idx1_b,
          rows0_a, rows0_b, rows1_a, rows1_b, out_v,
          sem0_a, sem0_b, sem1_a, sem1_b):
    wid = lax.axis_index("s") * NC + lax.axis_index("c")
    lane = lax.iota(jnp.int32, 16)
    lane_w = lane * OUT_W
    idx0 = (idx0_a, idx0_b)
    idx1 = (idx1_a, idx1_b)
    rows0 = (rows0_a, rows0_b)
    rows1 = (rows1_a, rows1_b)
    sems0 = (sem0_a, sem0_b)
    sems1 = (sem1_a, sem1_b)

    def make_build(lvl):
        res = np.float32(_RES[lvl])
        off = np.int32(lvl * TABLE_SZ)
        p = lvl % 2

        def build(g, c2):
            xv = xs_v[pl.ds(g * 16, 16)]
            yv = ys_v[pl.ds(g * 16, 16)]
            zv = zs_v[pl.ds(g * 16, 16)]
            spx = xv * res
            spy = yv * res
            spz = zv * res
            lx = spx.astype(jnp.int32)
            ly = spy.astype(jnp.int32)
            lz = spz.astype(jnp.int32)
            hx = jnp.where(spx > lx.astype(jnp.float32), lx + 1, lx)
            hy = jnp.where(spy > ly.astype(jnp.float32), ly + 1, ly)
            hz = jnp.where(spz > lz.astype(jnp.float32), lz + 1, lz)
            ax = (lx, hx)
            by = (ly * _C1, hy * _C1)
            cz = (lz * _C2, hz * _C2)
            gbase = g * 128
            for c, (i, j, k) in enumerate(_CORNERS):
                h = (((ax[i] ^ by[j] ^ cz[k]) & _MASK) + off) * 2
                idx0[p][pl.ds(gbase + c * 16, 16)] = h
                idx1[p][pl.ds(gbase + c * 16, 16)] = h + 1
            return c2

        return build

    def start_gather(lvl):
        p = lvl % 2
        lax.fori_loop(0, GROUPS, make_build(lvl), 0)
        cp0 = pltpu.async_copy(table.at[idx0[p]], rows0[p], sems0[p])
        cp1 = pltpu.async_copy(table.at[idx1[p]], rows1[p], sems1[p])
        return cp0, cp1

    def make_interp(lvl):
        p = lvl % 2

        def interp(g, c2):
            gp = g * 16
            xw = xs_v[pl.ds(gp, 16)]
            yw = ys_v[pl.ds(gp, 16)]
            zw = zs_v[pl.ds(gp, 16)]
            gbase = g * 128
            obase = gp * OUT_W + 2 * lvl
            for ch, rows in ((0, rows0[p]), (1, rows1[p])):
                f = [rows[pl.ds(gbase + c * 16, 16)] for c in range(8)]
                c00 = f[0] + xw * (f[1] - f[0])
                c01 = f[4] + xw * (f[5] - f[4])
                c10 = f[3] + xw * (f[2] - f[3])
                c11 = f[7] + xw * (f[6] - f[7])
                c0 = c00 + yw * (c10 - c00)
                c1 = c01 + yw * (c11 - c01)
                val = c0 + zw * (c1 - c0)
                plsc.store_scatter(out_v, [lane_w + (obase + ch)], val)
            return c2

        return interp

    def chunk_body(ci, carry):
        base = wid * PTS_PER_W + ci * CHUNK
        pltpu.sync_copy(x0.at[pl.ds(base, CHUNK)], xs_v)
        pltpu.sync_copy(x1.at[pl.ds(base, CHUNK)], ys_v)
        pltpu.sync_copy(x2.at[pl.ds(base, CHUNK)], zs_v)

        cps = start_gather(0)
        for lvl in range(NUM_LEVELS):
            nxt = start_gather(lvl + 1) if lvl + 1 < NUM_LEVELS else None
            cps[0].wait()
            cps[1].wait()
            lax.fori_loop(0, GROUPS, make_interp(lvl), 0)
            cps = nxt

        pltpu.sync_copy(out_v, out.at[pl.ds(base * OUT_W, CHUNK * OUT_W)])
        return carry

    lax.fori_loop(0, NCHUNK, chunk_body, 0)


_mesh = plsc.VectorSubcoreMesh(core_axis_name="c", subcore_axis_name="s")

_hash_enc = functools.partial(
    pl.kernel,
    out_type=jax.ShapeDtypeStruct((BATCH * OUT_W,), jnp.float32),
    mesh=_mesh,
    scratch_types=[
        pltpu.VMEM((CHUNK,), jnp.float32),
        pltpu.VMEM((CHUNK,), jnp.float32),
        pltpu.VMEM((CHUNK,), jnp.float32),
        pltpu.VMEM((CHUNK * 8,), jnp.int32),
        pltpu.VMEM((CHUNK * 8,), jnp.int32),
        pltpu.VMEM((CHUNK * 8,), jnp.int32),
        pltpu.VMEM((CHUNK * 8,), jnp.int32),
        pltpu.VMEM((CHUNK * 8,), jnp.float32),
        pltpu.VMEM((CHUNK * 8,), jnp.float32),
        pltpu.VMEM((CHUNK * 8,), jnp.float32),
        pltpu.VMEM((CHUNK * 8,), jnp.float32),
        pltpu.VMEM((CHUNK * OUT_W,), jnp.float32),
        pltpu.SemaphoreType.DMA,
        pltpu.SemaphoreType.DMA,
        pltpu.SemaphoreType.DMA,
        pltpu.SemaphoreType.DMA,
    ],
    compiler_params=pltpu.CompilerParams(needs_layout_passes=False),
)(_body)


def kernel(x, tables):
    xt = x.T
    table = tables.reshape(NUM_LEVELS * TABLE_SZ * FEATURE_DIM)
    flat = _hash_enc(xt[0], xt[1], xt[2], table)
    return flat.reshape(BATCH, OUT_W)
